# Initial kernel scaffold; baseline (speedup 1.0000x reference)
import jax
import jax.numpy as jnp
from jax.experimental import pallas as pl


def kernel(obs, adj, h0, c0, W_gcn, b_gcn, Wih, Whh, bih, bhh, Wa, ba):
    raise NotImplementedError("write your pallas kernel here")


# speedup vs baseline: 16.3977x; 16.3977x over previous
import functools

import jax
import jax.numpy as jnp
from jax import lax
from jax.experimental import pallas as pl
from jax.experimental.pallas import tpu as pltpu
from jax.experimental.pallas import tpu_sc as plsc

N = 10000
E = 320000
H = 128
NP = 10240
NTILES = 32
EPT = E // NTILES
EPT_PAD = 10240
NCHUNK = EPT_PAD // 128
ROWS_PT = NP // 16

_mesh = plsc.VectorSubcoreMesh(core_axis_name="c", subcore_axis_name="s")


@functools.partial(
    pl.kernel,
    mesh=_mesh,
    out_type=jax.ShapeDtypeStruct((2, NP), jnp.float32),
    scratch_types=[
        pltpu.VMEM((NCHUNK, 128), jnp.int32),
        pltpu.VMEM((128,), jnp.float32),
        pltpu.VMEM_SHARED((NP,), jnp.float32),
    ],
)
def _sc_degree(dst_idx_hbm, zeros_hbm, out_hbm, idx_v, ones_v, deg_sp):
    c = lax.axis_index("c")
    s = lax.axis_index("s")
    wid = c * 16 + s
    pltpu.sync_copy(dst_idx_hbm.at[wid], idx_v)
    pltpu.sync_copy(zeros_hbm.at[pl.ds(s * ROWS_PT, ROWS_PT)],
                    deg_sp.at[pl.ds(s * ROWS_PT, ROWS_PT)])
    for k in range(8):
        ones_v[pl.ds(k * 16, 16)] = jnp.full((16,), 1.0, jnp.float32)
    plsc.subcore_barrier()

    def body(j, carry):
        pltpu.sync_copy(ones_v, deg_sp.at[idx_v.at[j]], add=True)
        return carry

    lax.fori_loop(0, NCHUNK, body, 0)
    plsc.subcore_barrier()
    pltpu.sync_copy(deg_sp.at[pl.ds(s * ROWS_PT, ROWS_PT)],
                    out_hbm.at[c, pl.ds(s * ROWS_PT, ROWS_PT)])


@functools.partial(
    pl.kernel,
    mesh=_mesh,
    out_type=jax.ShapeDtypeStruct((2, NP, H), jnp.float32),
    scratch_types=[
        pltpu.VMEM((NCHUNK, 128), jnp.int32),
        pltpu.VMEM((NCHUNK, 128), jnp.int32),
        pltpu.VMEM((128, H), jnp.float32),
        pltpu.VMEM_SHARED((NP, H), jnp.float32),
        pltpu.SemaphoreType.DMA,
    ],
)
def _sc_messages(src_idx_hbm, dst_idx_hbm, y_hbm, out_hbm,
                 sidx_v, didx_v, rows_v, acc_sp, sem):
    c = lax.axis_index("c")
    s = lax.axis_index("s")
    wid = c * 16 + s
    pltpu.sync_copy(src_idx_hbm.at[wid], sidx_v)
    pltpu.sync_copy(dst_idx_hbm.at[wid], didx_v)
    pltpu.sync_copy(y_hbm.at[pl.ds(s * ROWS_PT, ROWS_PT)],
                    acc_sp.at[pl.ds(s * ROWS_PT, ROWS_PT)])
    plsc.subcore_barrier()

    def body(j, carry):
        pltpu.async_copy(y_hbm.at[sidx_v.at[j]], rows_v, sem).wait()
        pltpu.sync_copy(rows_v, acc_sp.at[didx_v.at[j]], add=True)
        return carry

    lax.fori_loop(0, NCHUNK, body, 0)
    plsc.subcore_barrier()
    pltpu.sync_copy(acc_sp.at[pl.ds(s * ROWS_PT, ROWS_PT)],
                    out_hbm.at[c, pl.ds(s * ROWS_PT, ROWS_PT), :])


def _y_body(obs_ref, w_ref, degt_ref, y_ref):
    deg = degt_ref[:, 0:1] + degt_ref[:, 1:2] + 1.0
    dinv = lax.rsqrt(deg)
    xw = jnp.dot(obs_ref[...], w_ref[...], preferred_element_type=jnp.float32)
    y_ref[...] = xw * dinv


def _g_body(acc_ref, y_ref, degt_ref, bg_ref, wih_ref, bsum_ref, g_ref):
    deg = degt_ref[:, 0:1] + degt_ref[:, 1:2] + 1.0
    dinv = lax.rsqrt(deg)
    x = dinv * (acc_ref[0] + acc_ref[1] - y_ref[...]) + bg_ref[...]
    g_ref[...] = jnp.dot(x, wih_ref[...],
                         preferred_element_type=jnp.float32) + bsum_ref[...]


LCHUNK = 1000
LGRID = N // LCHUNK


def _lstm_body(g_ref, whh_ref, wa_ref, ba_ref, h0_ref, c0_ref,
               act_ref, ht_ref, ct_ref, h_scr, c_scr, hs_scr):
    step = pl.program_id(0)

    @pl.when(step == 0)
    def _init():
        h_scr[...] = h0_ref[...]
        c_scr[...] = c0_ref[...]

    def body(t, carry):
        h, c = carry
        g = g_ref[pl.ds(t, 1), :]
        gates = g + jax.lax.dot_general(
            h, whh_ref[...], (((1,), (0,)), ((), ())),
            preferred_element_type=jnp.float32)
        i = jax.nn.sigmoid(gates[:, 0:H])
        f = jax.nn.sigmoid(gates[:, H:2 * H])
        gg = jnp.tanh(gates[:, 2 * H:3 * H])
        o = jax.nn.sigmoid(gates[:, 3 * H:4 * H])
        c2 = f * c + i * gg
        h2 = o * jnp.tanh(c2)
        hs_scr[pl.ds(t, 1), :] = h2
        return (h2, c2)

    hN, cN = lax.fori_loop(0, LCHUNK, body, (h_scr[...], c_scr[...]))
    h_scr[...] = hN
    c_scr[...] = cN
    act_ref[...] = jnp.dot(hs_scr[...], wa_ref[...],
                           preferred_element_type=jnp.float32) + ba_ref[...]
    ht_ref[...] = hN
    ct_ref[...] = cN


def kernel(obs, adj, h0, c0, W_gcn, b_gcn, Wih, Whh, bih, bhh, Wa, ba):
    f32 = jnp.float32
    srcr = adj[0].reshape(NTILES, EPT)
    dstr = adj[1].reshape(NTILES, EPT)
    npad = EPT_PAD - EPT
    pad_src = jnp.broadcast_to(
        (jnp.arange(npad, dtype=jnp.int32) % 128)[None, :], (NTILES, npad))
    pad_dst = jnp.broadcast_to(
        (N + jnp.arange(npad, dtype=jnp.int32))[None, :], (NTILES, npad))
    src_t = jnp.concatenate([srcr, pad_src], axis=1).reshape(NTILES, NCHUNK, 128)
    dst_t = jnp.concatenate([dstr, pad_dst], axis=1).reshape(NTILES, NCHUNK, 128)
    zeros_np = jnp.zeros((NP,), f32)

    deg_parts = _sc_degree(dst_t, zeros_np)
    deg_t = deg_parts.T

    RB = 1000
    grid_a = N // RB
    y = pl.pallas_call(
        _y_body,
        grid=(grid_a,),
        in_specs=[
            pl.BlockSpec((RB, H), lambda i: (i, 0)),
            pl.BlockSpec((H, H), lambda i: (0, 0)),
            pl.BlockSpec((RB, 2), lambda i: (i, 0)),
        ],
        out_specs=pl.BlockSpec((RB, H), lambda i: (i, 0)),
        out_shape=jax.ShapeDtypeStruct((NP, H), f32),
    )(obs, W_gcn, deg_t)

    acc = _sc_messages(src_t, dst_t, y)

    wih_t = Wih.T
    bsum = (bih + bhh)[None, :]
    bg2 = b_gcn[None, :]
    G = pl.pallas_call(
        _g_body,
        grid=(grid_a,),
        in_specs=[
            pl.BlockSpec((2, RB, H), lambda i: (0, i, 0)),
            pl.BlockSpec((RB, H), lambda i: (i, 0)),
            pl.BlockSpec((RB, 2), lambda i: (i, 0)),
            pl.BlockSpec((1, H), lambda i: (0, 0)),
            pl.BlockSpec((H, 4 * H), lambda i: (0, 0)),
            pl.BlockSpec((1, 4 * H), lambda i: (0, 0)),
        ],
        out_specs=pl.BlockSpec((RB, 4 * H), lambda i: (i, 0)),
        out_shape=jax.ShapeDtypeStruct((N, 4 * H), f32),
    )(acc, y, deg_t, bg2, wih_t, bsum)

    whh_t = Whh.T
    wa_t = Wa.T
    ba2 = ba[None, :]
    h0r = h0.reshape(1, H)
    c0r = c0.reshape(1, H)
    act, hT, cT = pl.pallas_call(
        _lstm_body,
        grid=(LGRID,),
        in_specs=[
            pl.BlockSpec((LCHUNK, 4 * H), lambda i: (i, 0)),
            pl.BlockSpec((H, 4 * H), lambda i: (0, 0)),
            pl.BlockSpec((H, 3), lambda i: (0, 0)),
            pl.BlockSpec((1, 3), lambda i: (0, 0)),
            pl.BlockSpec((1, H), lambda i: (0, 0)),
            pl.BlockSpec((1, H), lambda i: (0, 0)),
        ],
        out_specs=[
            pl.BlockSpec((LCHUNK, 3), lambda i: (i, 0)),
            pl.BlockSpec((1, H), lambda i: (0, 0)),
            pl.BlockSpec((1, H), lambda i: (0, 0)),
        ],
        out_shape=[
            jax.ShapeDtypeStruct((N, 3), f32),
            jax.ShapeDtypeStruct((1, H), f32),
            jax.ShapeDtypeStruct((1, H), f32),
        ],
        scratch_shapes=[
            pltpu.VMEM((1, H), f32),
            pltpu.VMEM((1, H), f32),
            pltpu.VMEM((LCHUNK, H), f32),
        ],
    )(G, whh_t, wa_t, ba2, h0r, c0r)

    return (act[None, :, :], hT[None, :, :], cT[None, :, :])


# speedup vs baseline: 16.8438x; 1.0272x over previous
import functools

import jax
import jax.numpy as jnp
from jax import lax
from jax.experimental import pallas as pl
from jax.experimental.pallas import tpu as pltpu
from jax.experimental.pallas import tpu_sc as plsc

N = 10000
E = 320000
H = 128
NP = 10240
NTILES = 32
EPT = E // NTILES
EPT_PAD = 10240
NCHUNK = EPT_PAD // 128
ROWS_PT = NP // 16

_mesh = plsc.VectorSubcoreMesh(core_axis_name="c", subcore_axis_name="s")


@functools.partial(
    pl.kernel,
    mesh=_mesh,
    out_type=jax.ShapeDtypeStruct((2, NP), jnp.float32),
    scratch_types=[
        pltpu.VMEM((NCHUNK, 128), jnp.int32),
        pltpu.VMEM((128,), jnp.float32),
        pltpu.VMEM_SHARED((NP,), jnp.float32),
    ],
)
def _sc_degree(dst_idx_hbm, zeros_hbm, out_hbm, idx_v, ones_v, deg_sp):
    c = lax.axis_index("c")
    s = lax.axis_index("s")
    wid = c * 16 + s
    pltpu.sync_copy(dst_idx_hbm.at[wid], idx_v)
    pltpu.sync_copy(zeros_hbm.at[pl.ds(s * ROWS_PT, ROWS_PT)],
                    deg_sp.at[pl.ds(s * ROWS_PT, ROWS_PT)])
    for k in range(8):
        ones_v[pl.ds(k * 16, 16)] = jnp.full((16,), 1.0, jnp.float32)
    plsc.subcore_barrier()

    def body(j, carry):
        pltpu.sync_copy(ones_v, deg_sp.at[idx_v.at[j]], add=True)
        return carry

    lax.fori_loop(0, NCHUNK, body, 0)
    plsc.subcore_barrier()
    pltpu.sync_copy(deg_sp.at[pl.ds(s * ROWS_PT, ROWS_PT)],
                    out_hbm.at[c, pl.ds(s * ROWS_PT, ROWS_PT)])


@functools.partial(
    pl.kernel,
    mesh=_mesh,
    out_type=jax.ShapeDtypeStruct((2, NP, H), jnp.float32),
    scratch_types=[
        pltpu.VMEM((NCHUNK, 128), jnp.int32),
        pltpu.VMEM((NCHUNK, 128), jnp.int32),
        pltpu.VMEM((128, H), jnp.float32),
        pltpu.VMEM_SHARED((NP, H), jnp.float32),
        pltpu.SemaphoreType.DMA,
    ],
)
def _sc_messages(src_idx_hbm, dst_idx_hbm, y_hbm, out_hbm,
                 sidx_v, didx_v, rows_v, acc_sp, sem):
    c = lax.axis_index("c")
    s = lax.axis_index("s")
    wid = c * 16 + s
    pltpu.sync_copy(src_idx_hbm.at[wid], sidx_v)
    pltpu.sync_copy(dst_idx_hbm.at[wid], didx_v)
    pltpu.sync_copy(y_hbm.at[pl.ds(s * ROWS_PT, ROWS_PT)],
                    acc_sp.at[pl.ds(s * ROWS_PT, ROWS_PT)])
    plsc.subcore_barrier()

    def body(j, carry):
        pltpu.async_copy(y_hbm.at[sidx_v.at[j]], rows_v, sem).wait()
        pltpu.sync_copy(rows_v, acc_sp.at[didx_v.at[j]], add=True)
        return carry

    lax.fori_loop(0, NCHUNK, body, 0)
    plsc.subcore_barrier()
    pltpu.sync_copy(acc_sp.at[pl.ds(s * ROWS_PT, ROWS_PT)],
                    out_hbm.at[c, pl.ds(s * ROWS_PT, ROWS_PT), :])


def _y_body(obs_ref, w_ref, degt_ref, y_ref):
    deg = degt_ref[:, 0:1] + degt_ref[:, 1:2] + 1.0
    dinv = lax.rsqrt(deg)
    xw = jnp.dot(obs_ref[...], w_ref[...], preferred_element_type=jnp.float32)
    y_ref[...] = xw * dinv


def _g_body(acc_ref, y_ref, degt_ref, bg_ref, wih_ref, bsum_ref, g_ref):
    deg = degt_ref[:, 0:1] + degt_ref[:, 1:2] + 1.0
    dinv = lax.rsqrt(deg)
    x = dinv * (acc_ref[0] + acc_ref[1] - y_ref[...]) + bg_ref[...]
    g_ref[...] = jnp.dot(x, wih_ref[...],
                         preferred_element_type=jnp.float32) + bsum_ref[...]


LCHUNK = 1000
LGRID = N // LCHUNK


UNROLL = 8


def _lstm_body(g_ref, whh_ref, wa_ref, ba_ref, h0_ref, c0_ref,
               act_ref, ht_ref, ct_ref, h_scr, c_scr, hs_scr):
    step = pl.program_id(0)

    @pl.when(step == 0)
    def _init():
        h_scr[...] = h0_ref[...]
        c_scr[...] = c0_ref[...]

    whh = whh_ref[...]

    def body(jo, carry):
        h, c = carry
        base = jo * UNROLL
        g8 = g_ref[pl.ds(base, UNROLL), :]
        rows = []
        for k in range(UNROLL):
            hb = h.astype(jnp.bfloat16)
            gates = g8[k:k + 1, :] + jax.lax.dot_general(
                hb, whh, (((1,), (0,)), ((), ())),
                preferred_element_type=jnp.float32)
            sg = jax.nn.sigmoid(gates)
            i = sg[:, 0:H]
            f = sg[:, H:2 * H]
            gp = sg[:, 2 * H:3 * H] + sg[:, 2 * H:3 * H] - 1.0
            o = sg[:, 3 * H:4 * H]
            c = f * c + i * gp
            h = o * (jax.nn.sigmoid(c + c) * 2.0 - 1.0)
            rows.append(h)
        hs_scr[pl.ds(base, UNROLL), :] = jnp.concatenate(rows, axis=0)
        return (h, c)

    hN, cN = lax.fori_loop(0, LCHUNK // UNROLL, body,
                           (h_scr[...], c_scr[...]))
    h_scr[...] = hN
    c_scr[...] = cN
    act_ref[...] = jnp.dot(hs_scr[...], wa_ref[...],
                           preferred_element_type=jnp.float32) + ba_ref[...]
    ht_ref[...] = hN
    ct_ref[...] = cN


def kernel(obs, adj, h0, c0, W_gcn, b_gcn, Wih, Whh, bih, bhh, Wa, ba):
    f32 = jnp.float32
    srcr = adj[0].reshape(NTILES, EPT)
    dstr = adj[1].reshape(NTILES, EPT)
    npad = EPT_PAD - EPT
    pad_src = jnp.broadcast_to(
        (jnp.arange(npad, dtype=jnp.int32) % 128)[None, :], (NTILES, npad))
    pad_dst = jnp.broadcast_to(
        (N + jnp.arange(npad, dtype=jnp.int32))[None, :], (NTILES, npad))
    src_t = jnp.concatenate([srcr, pad_src], axis=1).reshape(NTILES, NCHUNK, 128)
    dst_t = jnp.concatenate([dstr, pad_dst], axis=1).reshape(NTILES, NCHUNK, 128)
    zeros_np = jnp.zeros((NP,), f32)

    deg_parts = _sc_degree(dst_t, zeros_np)
    deg_t = deg_parts.T

    RB = 1000
    grid_a = N // RB
    y = pl.pallas_call(
        _y_body,
        grid=(grid_a,),
        in_specs=[
            pl.BlockSpec((RB, H), lambda i: (i, 0)),
            pl.BlockSpec((H, H), lambda i: (0, 0)),
            pl.BlockSpec((RB, 2), lambda i: (i, 0)),
        ],
        out_specs=pl.BlockSpec((RB, H), lambda i: (i, 0)),
        out_shape=jax.ShapeDtypeStruct((NP, H), f32),
    )(obs, W_gcn, deg_t)

    acc = _sc_messages(src_t, dst_t, y)

    gscale = jnp.concatenate([jnp.ones((2 * H,), f32),
                              jnp.full((H,), 2.0, f32),
                              jnp.ones((H,), f32)])[None, :]
    wih_t = Wih.T * gscale
    bsum = ((bih + bhh)[None, :]) * gscale
    bg2 = b_gcn[None, :]
    G = pl.pallas_call(
        _g_body,
        grid=(grid_a,),
        in_specs=[
            pl.BlockSpec((2, RB, H), lambda i: (0, i, 0)),
            pl.BlockSpec((RB, H), lambda i: (i, 0)),
            pl.BlockSpec((RB, 2), lambda i: (i, 0)),
            pl.BlockSpec((1, H), lambda i: (0, 0)),
            pl.BlockSpec((H, 4 * H), lambda i: (0, 0)),
            pl.BlockSpec((1, 4 * H), lambda i: (0, 0)),
        ],
        out_specs=pl.BlockSpec((RB, 4 * H), lambda i: (i, 0)),
        out_shape=jax.ShapeDtypeStruct((N, 4 * H), f32),
    )(acc, y, deg_t, bg2, wih_t, bsum)

    whh_t = (Whh.T * gscale).astype(jnp.bfloat16)
    wa_t = Wa.T
    ba2 = ba[None, :]
    h0r = h0.reshape(1, H)
    c0r = c0.reshape(1, H)
    act, hT, cT = pl.pallas_call(
        _lstm_body,
        grid=(LGRID,),
        in_specs=[
            pl.BlockSpec((LCHUNK, 4 * H), lambda i: (i, 0)),
            pl.BlockSpec((H, 4 * H), lambda i: (0, 0)),
            pl.BlockSpec((H, 3), lambda i: (0, 0)),
            pl.BlockSpec((1, 3), lambda i: (0, 0)),
            pl.BlockSpec((1, H), lambda i: (0, 0)),
            pl.BlockSpec((1, H), lambda i: (0, 0)),
        ],
        out_specs=[
            pl.BlockSpec((LCHUNK, 3), lambda i: (i, 0)),
            pl.BlockSpec((1, H), lambda i: (0, 0)),
            pl.BlockSpec((1, H), lambda i: (0, 0)),
        ],
        out_shape=[
            jax.ShapeDtypeStruct((N, 3), f32),
            jax.ShapeDtypeStruct((1, H), f32),
            jax.ShapeDtypeStruct((1, H), f32),
        ],
        scratch_shapes=[
            pltpu.VMEM((1, H), f32),
            pltpu.VMEM((1, H), f32),
            pltpu.VMEM((LCHUNK, H), f32),
        ],
    )(G, whh_t, wa_t, ba2, h0r, c0r)

    return (act[None, :, :], hT[None, :, :], cT[None, :, :])
